# Initial kernel scaffold; baseline (speedup 1.0000x reference)
import jax
import jax.numpy as jnp
from jax.experimental import pallas as pl


def kernel(feats, edge_index, W0, W1, W2, W3, W4, b0, b1, b2, b3, b4, W_out, b_out):
    raise NotImplementedError("write your pallas kernel here")


# speedup vs baseline: 21.1845x; 21.1845x over previous
import dataclasses
import functools

import jax
import jax.numpy as jnp
from jax import lax
from jax.experimental import pallas as pl
from jax.experimental.pallas import tpu as pltpu
from jax.experimental.pallas import tpu_sc as plsc

_NC = 2
_NS = 16
_NW = _NC * _NS


def _sc_params():
    cp = pltpu.CompilerParams(use_tc_tiling_on_sc=False)
    if "needs_layout_passes" in pltpu.CompilerParams.__dataclass_fields__:
        cp = dataclasses.replace(cp, needs_layout_passes=False)
    return cp


def _agg_pass(table, eidx3, nbuf=2, lag=1):
    n, c = table.shape
    chunk = eidx3.shape[2]
    e = eidx3.shape[1] * chunk
    ept = e // _NW
    nch = ept // chunk
    assert nch * chunk == ept
    npt = (n // _NS) & ~7
    tail = n - npt * _NS
    zrows = npt // 3
    assert tail % 8 == 0 and zrows % 8 == 0 and zrows * 3 == npt
    assert tail <= zrows
    mesh = plsc.VectorSubcoreMesh(core_axis_name="c", subcore_axis_name="s")

    @functools.partial(
        pl.kernel,
        out_type=jax.ShapeDtypeStruct((_NC, n, c), jnp.float32),
        mesh=mesh,
        compiler_params=_sc_params(),
        scratch_types=[
            pltpu.VMEM((nch, chunk), jnp.int32),
            pltpu.VMEM((nch, chunk), jnp.int32),
            [pltpu.VMEM((chunk, c), jnp.float32)] * nbuf,
            pltpu.VMEM((zrows, c), jnp.float32),
            pltpu.VMEM_SHARED((n, c), jnp.float32),
            [pltpu.SemaphoreType.DMA] * nbuf,
            [pltpu.SemaphoreType.DMA] * nbuf,
        ],
    )
    def kern(tbl_h, ei_h, out_h, srcv, dstv, rows, zbuf, acc, gsem, ssem):
        cid = lax.axis_index("c")
        sid = lax.axis_index("s")
        wid = cid * _NS + sid
        off = pl.multiple_of(sid * npt, 8)
        ibase = wid * nch

        pltpu.sync_copy(ei_h.at[0, pl.ds(ibase, nch)], srcv)
        pltpu.sync_copy(ei_h.at[1, pl.ds(ibase, nch)], dstv)

        def gather(k):
            j = k % nbuf
            pltpu.async_copy(tbl_h.at[srcv.at[k]], rows[j], gsem[j])

        def gather_wait(k):
            j = k % nbuf
            pltpu.make_async_copy(tbl_h.at[srcv.at[k]], rows[j],
                                  gsem[j]).wait()

        def scatter(k):
            j = k % nbuf
            pltpu.async_copy(rows[j], acc.at[dstv.at[k]], ssem[j], add=True)

        def scatter_wait(k):
            j = k % nbuf
            pltpu.make_async_copy(rows[j], acc.at[dstv.at[k]],
                                  ssem[j]).wait()

        for k in range(min(lag, nch)):
            gather(k)

        @pl.loop(0, zrows)
        def _zero(i):
            for j in range(c // 16):
                zbuf[i, pl.ds(j * 16, 16)] = jnp.zeros((16,), jnp.float32)

        for z in range(3):
            pltpu.sync_copy(zbuf,
                            acc.at[pl.ds(pl.multiple_of(off + z * zrows, 8),
                                         zrows)])

        @pl.when(sid == _NS - 1)
        def _zero_tail():
            pltpu.sync_copy(zbuf.at[pl.ds(0, tail)],
                            acc.at[pl.ds(npt * _NS, tail)])

        plsc.subcore_barrier()

        for k in range(lag, nch + lag):
            if k < nch:
                if k >= nbuf:
                    scatter_wait(k - nbuf)
                gather(k)
            kp = k - lag
            gather_wait(kp)
            scatter(kp)
        for kp in range(max(nch - nbuf, 0), nch):
            scatter_wait(kp)

        plsc.subcore_barrier()
        pltpu.sync_copy(acc.at[pl.ds(off, npt)],
                        out_h.at[cid, pl.ds(off, npt)])

        @pl.when(sid == _NS - 1)
        def _out_tail():
            pltpu.sync_copy(acc.at[pl.ds(npt * _NS, tail)],
                            out_h.at[cid, pl.ds(npt * _NS, tail)])

    return kern(table, eidx3)


def _degree_pass(eidx3, n):
    chunk = eidx3.shape[2]
    e = eidx3.shape[1] * chunk
    ept = e // _NW
    nch = ept // chunk
    assert nch * chunk == ept
    npt = (n // _NS) & ~7
    tail = n - npt * _NS
    zrows = npt // 3
    assert tail % 8 == 0 and zrows % 8 == 0 and zrows * 3 == npt
    assert tail <= zrows
    mesh = plsc.VectorSubcoreMesh(core_axis_name="c", subcore_axis_name="s")

    @functools.partial(
        pl.kernel,
        out_type=jax.ShapeDtypeStruct((_NC, n, 16), jnp.float32),
        mesh=mesh,
        compiler_params=_sc_params(),
        scratch_types=[
            pltpu.VMEM((nch, chunk), jnp.int32),
            pltpu.VMEM((nch, chunk), jnp.int32),
            pltpu.VMEM((chunk, 16), jnp.float32),
            pltpu.VMEM((chunk, 16), jnp.float32),
            pltpu.VMEM((zrows, 16), jnp.float32),
            pltpu.VMEM_SHARED((n, 16), jnp.float32),
            pltpu.SemaphoreType.DMA,
            pltpu.SemaphoreType.DMA,
        ],
    )
    def kern(ei_h, out_h, srcv, dstv, ones_s, ones_d, zbuf, acc, sem0, sem1):
        cid = lax.axis_index("c")
        sid = lax.axis_index("s")
        wid = cid * _NS + sid
        off = pl.multiple_of(sid * npt, 8)
        ibase = wid * nch

        pltpu.sync_copy(ei_h.at[0, pl.ds(ibase, nch)], srcv)
        pltpu.sync_copy(ei_h.at[1, pl.ds(ibase, nch)], dstv)

        lo = (lax.iota(jnp.int32, 16) < 8).astype(jnp.float32)

        @pl.loop(0, chunk)
        def _fill(i):
            ones_s[i, pl.ds(0, 16)] = lo
            ones_d[i, pl.ds(0, 16)] = 1.0 - lo

        @pl.loop(0, zrows)
        def _zero(i):
            zbuf[i, pl.ds(0, 16)] = jnp.zeros((16,), jnp.float32)

        for z in range(3):
            pltpu.sync_copy(zbuf,
                            acc.at[pl.ds(pl.multiple_of(off + z * zrows, 8),
                                         zrows)])

        @pl.when(sid == _NS - 1)
        def _zero_tail():
            pltpu.sync_copy(zbuf.at[pl.ds(0, tail)],
                            acc.at[pl.ds(npt * _NS, tail)])

        plsc.subcore_barrier()

        for k in range(nch):
            pltpu.async_copy(ones_s, acc.at[srcv.at[k]], sem0, add=True)
            pltpu.async_copy(ones_d, acc.at[dstv.at[k]], sem1, add=True)
        for k in range(nch):
            pltpu.make_async_copy(ones_s, acc.at[srcv.at[k]], sem0).wait()
            pltpu.make_async_copy(ones_d, acc.at[dstv.at[k]], sem1).wait()

        plsc.subcore_barrier()
        pltpu.sync_copy(acc.at[pl.ds(off, npt)],
                        out_h.at[cid, pl.ds(off, npt)])

        @pl.when(sid == _NS - 1)
        def _out_tail():
            pltpu.sync_copy(acc.at[pl.ds(npt * _NS, tail)],
                            out_h.at[cid, pl.ds(npt * _NS, tail)])

    return kern(eidx3)


_HIGH = jax.lax.Precision.HIGHEST


def _k0(feats, w0, degp):
    n = feats.shape[0]
    h = w0.shape[1]

    def body(f_ref, w_ref, d_ref, t0_ref, ns_ref, nd_ref):
        deg = d_ref[0] + d_ref[1]
        deg_s = jnp.broadcast_to(deg[:, 0:1], deg.shape)
        deg_d = jnp.broadcast_to(deg[:, 8:9], deg.shape)
        ns = lax.rsqrt(jnp.maximum(deg_s, 1.0))
        nd = lax.rsqrt(jnp.maximum(deg_d, 1.0))
        hw = jnp.dot(f_ref[...], w_ref[...], precision=_HIGH,
                     preferred_element_type=jnp.float32)
        t0_ref[...] = hw * ns
        ns_ref[...] = ns
        nd_ref[...] = nd

    return pl.pallas_call(
        body,
        out_shape=(
            jax.ShapeDtypeStruct((n, h), jnp.float32),
            jax.ShapeDtypeStruct((n, 16), jnp.float32),
            jax.ShapeDtypeStruct((n, 16), jnp.float32),
        ),
    )(feats, w0, degp)


def _klayer(aggp, ndst, nsrc, b, w):
    n = aggp.shape[1]

    def body(a_ref, nd_ref, ns_ref, b_ref, w_ref, t_ref):
        agg = a_ref[0, :, 0:16] + a_ref[1, :, 0:16]
        hcur = jnp.maximum(agg * nd_ref[...] + b_ref[...], 0.0)
        hw = jnp.dot(hcur, w_ref[...], precision=_HIGH,
                     preferred_element_type=jnp.float32) * ns_ref[...]
        t_ref[:, 0:16] = hw
        t_ref[:, 16:32] = hcur

    return pl.pallas_call(
        body,
        out_shape=jax.ShapeDtypeStruct((n, 32), jnp.float32),
    )(aggp, ndst, nsrc, b, w)


def _klast(aggp, ndst, b):
    n = aggp.shape[1]

    def body(a_ref, nd_ref, b_ref, t_ref):
        agg = a_ref[0, :, 0:16] + a_ref[1, :, 0:16]
        t_ref[...] = jnp.maximum(agg * nd_ref[...] + b_ref[...], 0.0)

    return pl.pallas_call(
        body,
        out_shape=jax.ShapeDtypeStruct((n, 16), jnp.float32),
    )(aggp, ndst, b)


def _kout(a1, a2, a3, a4, a5, w_out, b_out, bn=2000):
    n = a1.shape[1]
    out_d = w_out.shape[1]
    assert n % bn == 0

    def body(a1r, a2r, a3r, a4r, a5r, w_ref, b_ref, o_ref):
        h2 = jnp.concatenate(
            [a1r[0, :, 16:32] + a1r[1, :, 16:32],
             a2r[0, :, 16:32] + a2r[1, :, 16:32],
             a3r[0, :, 16:32] + a3r[1, :, 16:32],
             a4r[0, :, 16:32] + a4r[1, :, 16:32],
             a5r[0] + a5r[1]],
            axis=1)
        o_ref[...] = jnp.dot(h2, w_ref[...], precision=_HIGH,
                             preferred_element_type=jnp.float32) + b_ref[...]

    blk32 = pl.BlockSpec((2, bn, 32), lambda i: (0, i, 0))
    blk16 = pl.BlockSpec((2, bn, 16), lambda i: (0, i, 0))
    return pl.pallas_call(
        body,
        grid=(n // bn,),
        in_specs=[blk32, blk32, blk32, blk32, blk16,
                  pl.BlockSpec(w_out.shape, lambda i: (0, 0)),
                  pl.BlockSpec(b_out.shape, lambda i: (0, 0))],
        out_specs=pl.BlockSpec((bn, out_d), lambda i: (i, 0)),
        out_shape=jax.ShapeDtypeStruct((n, out_d), jnp.float32),
    )(a1, a2, a3, a4, a5, w_out, b_out)


def kernel(feats, edge_index, W0, W1, W2, W3, W4, b0, b1, b2, b3, b4,
           W_out, b_out):
    n = feats.shape[0]
    e = edge_index.shape[1]
    ei_a = edge_index.reshape(2, e // 2500, 2500)
    ei_b = edge_index.reshape(2, e // 1250, 1250)

    degp = _degree_pass(ei_a, n)

    t0, nsrc, ndst = _k0(feats, W0, degp)
    a0 = _agg_pass(t0, ei_a)

    b0r = b0.reshape(1, 16)
    b1r = b1.reshape(1, 16)
    b2r = b2.reshape(1, 16)
    b3r = b3.reshape(1, 16)
    b4r = b4.reshape(1, 16)

    t1 = _klayer(a0, ndst, nsrc, b0r, W1)
    a1 = _agg_pass(t1, ei_b)
    t2 = _klayer(a1, ndst, nsrc, b1r, W2)
    a2 = _agg_pass(t2, ei_b)
    t3 = _klayer(a2, ndst, nsrc, b2r, W3)
    a3 = _agg_pass(t3, ei_b)
    t4 = _klayer(a3, ndst, nsrc, b3r, W4)
    a4 = _agg_pass(t4, ei_b)
    t5 = _klast(a4, ndst, b4r)
    a5 = _agg_pass(t5, ei_a)

    return _kout(a1, a2, a3, a4, a5, W_out, b_out.reshape(1, -1))


# speedup vs baseline: 21.7171x; 1.0251x over previous
import dataclasses
import functools

import jax
import jax.numpy as jnp
from jax import lax
from jax.experimental import pallas as pl
from jax.experimental.pallas import tpu as pltpu
from jax.experimental.pallas import tpu_sc as plsc

_NC = 2
_NS = 16
_NW = _NC * _NS


def _sc_params():
    cp = pltpu.CompilerParams(use_tc_tiling_on_sc=False)
    if "needs_layout_passes" in pltpu.CompilerParams.__dataclass_fields__:
        cp = dataclasses.replace(cp, needs_layout_passes=False)
    return cp


def _agg_pass(table, eidx3, nbuf=4, lag=2):
    n, c = table.shape
    chunk = eidx3.shape[2]
    e = eidx3.shape[1] * chunk
    ept = e // _NW
    nch = ept // chunk
    assert nch * chunk == ept
    npt = (n // _NS) & ~7
    tail = n - npt * _NS
    zrows = npt // 3
    assert tail % 8 == 0 and zrows % 8 == 0 and zrows * 3 == npt
    assert tail <= zrows
    mesh = plsc.VectorSubcoreMesh(core_axis_name="c", subcore_axis_name="s")

    @functools.partial(
        pl.kernel,
        out_type=jax.ShapeDtypeStruct((_NC, n, c), jnp.float32),
        mesh=mesh,
        compiler_params=_sc_params(),
        scratch_types=[
            pltpu.VMEM((nch, chunk), jnp.int32),
            pltpu.VMEM((nch, chunk), jnp.int32),
            [pltpu.VMEM((chunk, c), jnp.float32)] * nbuf,
            pltpu.VMEM((zrows, c), jnp.float32),
            pltpu.VMEM_SHARED((n, c), jnp.float32),
            [pltpu.SemaphoreType.DMA] * nbuf,
            [pltpu.SemaphoreType.DMA] * nbuf,
        ],
    )
    def kern(tbl_h, ei_h, out_h, srcv, dstv, rows, zbuf, acc, gsem, ssem):
        cid = lax.axis_index("c")
        sid = lax.axis_index("s")
        wid = cid * _NS + sid
        off = pl.multiple_of(sid * npt, 8)
        ibase = wid * nch

        pltpu.sync_copy(ei_h.at[0, pl.ds(ibase, nch)], srcv)
        pltpu.sync_copy(ei_h.at[1, pl.ds(ibase, nch)], dstv)

        def gather(k):
            j = k % nbuf
            pltpu.async_copy(tbl_h.at[srcv.at[k]], rows[j], gsem[j])

        def gather_wait(k):
            j = k % nbuf
            pltpu.make_async_copy(tbl_h.at[srcv.at[k]], rows[j],
                                  gsem[j]).wait()

        def scatter(k):
            j = k % nbuf
            pltpu.async_copy(rows[j], acc.at[dstv.at[k]], ssem[j], add=True)

        def scatter_wait(k):
            j = k % nbuf
            pltpu.make_async_copy(rows[j], acc.at[dstv.at[k]],
                                  ssem[j]).wait()

        for k in range(min(lag, nch)):
            gather(k)

        @pl.loop(0, zrows)
        def _zero(i):
            for j in range(c // 16):
                zbuf[i, pl.ds(j * 16, 16)] = jnp.zeros((16,), jnp.float32)

        for z in range(3):
            pltpu.sync_copy(zbuf,
                            acc.at[pl.ds(pl.multiple_of(off + z * zrows, 8),
                                         zrows)])

        @pl.when(sid == _NS - 1)
        def _zero_tail():
            pltpu.sync_copy(zbuf.at[pl.ds(0, tail)],
                            acc.at[pl.ds(npt * _NS, tail)])

        plsc.subcore_barrier()

        for k in range(lag, nch + lag):
            if k < nch:
                if k >= nbuf:
                    scatter_wait(k - nbuf)
                gather(k)
            kp = k - lag
            gather_wait(kp)
            scatter(kp)
        for kp in range(max(nch - nbuf, 0), nch):
            scatter_wait(kp)

        plsc.subcore_barrier()
        pltpu.sync_copy(acc.at[pl.ds(off, npt)],
                        out_h.at[cid, pl.ds(off, npt)])

        @pl.when(sid == _NS - 1)
        def _out_tail():
            pltpu.sync_copy(acc.at[pl.ds(npt * _NS, tail)],
                            out_h.at[cid, pl.ds(npt * _NS, tail)])

    return kern(table, eidx3)


def _degree_pass(eidx3, n):
    chunk = eidx3.shape[2]
    e = eidx3.shape[1] * chunk
    ept = e // _NW
    nch = ept // chunk
    assert nch * chunk == ept
    npt = (n // _NS) & ~7
    tail = n - npt * _NS
    zrows = npt // 3
    assert tail % 8 == 0 and zrows % 8 == 0 and zrows * 3 == npt
    assert tail <= zrows
    mesh = plsc.VectorSubcoreMesh(core_axis_name="c", subcore_axis_name="s")

    @functools.partial(
        pl.kernel,
        out_type=jax.ShapeDtypeStruct((_NC, n, 16), jnp.float32),
        mesh=mesh,
        compiler_params=_sc_params(),
        scratch_types=[
            pltpu.VMEM((nch, chunk), jnp.int32),
            pltpu.VMEM((nch, chunk), jnp.int32),
            pltpu.VMEM((chunk, 16), jnp.float32),
            pltpu.VMEM((chunk, 16), jnp.float32),
            pltpu.VMEM((zrows, 16), jnp.float32),
            pltpu.VMEM_SHARED((n, 16), jnp.float32),
            pltpu.SemaphoreType.DMA,
            pltpu.SemaphoreType.DMA,
        ],
    )
    def kern(ei_h, out_h, srcv, dstv, ones_s, ones_d, zbuf, acc, sem0, sem1):
        cid = lax.axis_index("c")
        sid = lax.axis_index("s")
        wid = cid * _NS + sid
        off = pl.multiple_of(sid * npt, 8)
        ibase = wid * nch

        pltpu.sync_copy(ei_h.at[0, pl.ds(ibase, nch)], srcv)
        pltpu.sync_copy(ei_h.at[1, pl.ds(ibase, nch)], dstv)

        lo = (lax.iota(jnp.int32, 16) < 8).astype(jnp.float32)

        @pl.loop(0, chunk)
        def _fill(i):
            ones_s[i, pl.ds(0, 16)] = lo
            ones_d[i, pl.ds(0, 16)] = 1.0 - lo

        @pl.loop(0, zrows)
        def _zero(i):
            zbuf[i, pl.ds(0, 16)] = jnp.zeros((16,), jnp.float32)

        for z in range(3):
            pltpu.sync_copy(zbuf,
                            acc.at[pl.ds(pl.multiple_of(off + z * zrows, 8),
                                         zrows)])

        @pl.when(sid == _NS - 1)
        def _zero_tail():
            pltpu.sync_copy(zbuf.at[pl.ds(0, tail)],
                            acc.at[pl.ds(npt * _NS, tail)])

        plsc.subcore_barrier()

        for k in range(nch):
            pltpu.async_copy(ones_s, acc.at[srcv.at[k]], sem0, add=True)
            pltpu.async_copy(ones_d, acc.at[dstv.at[k]], sem1, add=True)
        for k in range(nch):
            pltpu.make_async_copy(ones_s, acc.at[srcv.at[k]], sem0).wait()
            pltpu.make_async_copy(ones_d, acc.at[dstv.at[k]], sem1).wait()

        plsc.subcore_barrier()
        pltpu.sync_copy(acc.at[pl.ds(off, npt)],
                        out_h.at[cid, pl.ds(off, npt)])

        @pl.when(sid == _NS - 1)
        def _out_tail():
            pltpu.sync_copy(acc.at[pl.ds(npt * _NS, tail)],
                            out_h.at[cid, pl.ds(npt * _NS, tail)])

    return kern(eidx3)


_HIGH = jax.lax.Precision.HIGHEST


def _k0(feats, w0, degp):
    n = feats.shape[0]
    h = w0.shape[1]

    def body(f_ref, w_ref, d_ref, t0_ref, ns_ref, nd_ref):
        deg = d_ref[0] + d_ref[1]
        deg_s = jnp.broadcast_to(deg[:, 0:1], deg.shape)
        deg_d = jnp.broadcast_to(deg[:, 8:9], deg.shape)
        ns = lax.rsqrt(jnp.maximum(deg_s, 1.0))
        nd = lax.rsqrt(jnp.maximum(deg_d, 1.0))
        hw = jnp.dot(f_ref[...], w_ref[...], precision=_HIGH,
                     preferred_element_type=jnp.float32)
        t0_ref[...] = hw * ns
        ns_ref[...] = ns
        nd_ref[...] = nd

    return pl.pallas_call(
        body,
        out_shape=(
            jax.ShapeDtypeStruct((n, h), jnp.float32),
            jax.ShapeDtypeStruct((n, 16), jnp.float32),
            jax.ShapeDtypeStruct((n, 16), jnp.float32),
        ),
    )(feats, w0, degp)


def _klayer(aggp, ndst, nsrc, b, w):
    n = aggp.shape[1]

    def body(a_ref, nd_ref, ns_ref, b_ref, w_ref, t_ref):
        agg = a_ref[0, :, 0:16] + a_ref[1, :, 0:16]
        hcur = jnp.maximum(agg * nd_ref[...] + b_ref[...], 0.0)
        hw = jnp.dot(hcur, w_ref[...], precision=_HIGH,
                     preferred_element_type=jnp.float32) * ns_ref[...]
        t_ref[:, 0:16] = hw
        t_ref[:, 16:32] = hcur

    return pl.pallas_call(
        body,
        out_shape=jax.ShapeDtypeStruct((n, 32), jnp.float32),
    )(aggp, ndst, nsrc, b, w)


def _klast(aggp, ndst, b):
    n = aggp.shape[1]

    def body(a_ref, nd_ref, b_ref, t_ref):
        agg = a_ref[0, :, 0:16] + a_ref[1, :, 0:16]
        t_ref[...] = jnp.maximum(agg * nd_ref[...] + b_ref[...], 0.0)

    return pl.pallas_call(
        body,
        out_shape=jax.ShapeDtypeStruct((n, 16), jnp.float32),
    )(aggp, ndst, b)


def _kout(a1, a2, a3, a4, a5, w_out, b_out, bn=2000):
    n = a1.shape[1]
    out_d = w_out.shape[1]
    assert n % bn == 0

    def body(a1r, a2r, a3r, a4r, a5r, w_ref, b_ref, o_ref):
        h2 = jnp.concatenate(
            [a1r[0, :, 16:32] + a1r[1, :, 16:32],
             a2r[0, :, 16:32] + a2r[1, :, 16:32],
             a3r[0, :, 16:32] + a3r[1, :, 16:32],
             a4r[0, :, 16:32] + a4r[1, :, 16:32],
             a5r[0] + a5r[1]],
            axis=1)
        o_ref[...] = jnp.dot(h2, w_ref[...], precision=_HIGH,
                             preferred_element_type=jnp.float32) + b_ref[...]

    blk32 = pl.BlockSpec((2, bn, 32), lambda i: (0, i, 0))
    blk16 = pl.BlockSpec((2, bn, 16), lambda i: (0, i, 0))
    return pl.pallas_call(
        body,
        grid=(n // bn,),
        in_specs=[blk32, blk32, blk32, blk32, blk16,
                  pl.BlockSpec(w_out.shape, lambda i: (0, 0)),
                  pl.BlockSpec(b_out.shape, lambda i: (0, 0))],
        out_specs=pl.BlockSpec((bn, out_d), lambda i: (i, 0)),
        out_shape=jax.ShapeDtypeStruct((n, out_d), jnp.float32),
    )(a1, a2, a3, a4, a5, w_out, b_out)


def kernel(feats, edge_index, W0, W1, W2, W3, W4, b0, b1, b2, b3, b4,
           W_out, b_out):
    n = feats.shape[0]
    e = edge_index.shape[1]
    ei_a = edge_index.reshape(2, e // 1000, 1000)
    ei_b = edge_index.reshape(2, e // 625, 625)

    degp = _degree_pass(ei_a, n)

    t0, nsrc, ndst = _k0(feats, W0, degp)
    a0 = _agg_pass(t0, ei_a)

    b0r = b0.reshape(1, 16)
    b1r = b1.reshape(1, 16)
    b2r = b2.reshape(1, 16)
    b3r = b3.reshape(1, 16)
    b4r = b4.reshape(1, 16)

    t1 = _klayer(a0, ndst, nsrc, b0r, W1)
    a1 = _agg_pass(t1, ei_b)
    t2 = _klayer(a1, ndst, nsrc, b1r, W2)
    a2 = _agg_pass(t2, ei_b)
    t3 = _klayer(a2, ndst, nsrc, b2r, W3)
    a3 = _agg_pass(t3, ei_b)
    t4 = _klayer(a3, ndst, nsrc, b3r, W4)
    a4 = _agg_pass(t4, ei_b)
    t5 = _klast(a4, ndst, b4r)
    a5 = _agg_pass(t5, ei_a)

    return _kout(a1, a2, a3, a4, a5, W_out, b_out.reshape(1, -1))
